# Initial kernel scaffold; baseline (speedup 1.0000x reference)
import functools

import jax
import jax.numpy as jnp
from jax import lax
from jax.experimental import pallas as pl
from jax.experimental.pallas import tpu as pltpu
from jax.experimental.pallas import tpu_sc as plsc

N = 10000
E = 320000
D = 128
G = 64
NC = 2
NS = 16
NW = NC * NS
EPT = E // NW
K = 80
NITER = EPT // K
NPAD = 10240
RPT = NPAD // NS
DW = 16

f32 = jnp.float32
_mesh = plsc.VectorSubcoreMesh(core_axis_name="c", subcore_axis_name="s")


@functools.partial(
    pl.kernel,
    out_type=jax.ShapeDtypeStruct((NC, NPAD, DW), f32),
    mesh=_mesh,
    scratch_types=[
        pltpu.VMEM((K,), jnp.int32),
        pltpu.VMEM((K, DW), f32),
        pltpu.VMEM((128, DW), f32),
        pltpu.VMEM_SHARED((NPAD, DW), f32),
    ],
)
def _deg_kernel(dst_hbm, out_hbm, dst_v, ones_v, zbuf, acc):
    c = lax.axis_index("c")
    s = lax.axis_index("s")
    ones = jnp.full((16,), 1.0, f32)
    zeros = jnp.zeros((16,), f32)

    def fill(i, _):
        ones_v[i, :] = ones
        zbuf[i, :] = zeros
        return 0

    lax.fori_loop(0, K, fill, 0)

    def fill2(i, _):
        zbuf[i, :] = zeros
        return 0

    lax.fori_loop(K, 128, fill2, 0)
    for k in range(RPT // 128):
        pltpu.sync_copy(zbuf, acc.at[pl.ds(s * RPT + k * 128, 128)])
    plsc.subcore_barrier()

    base = (c * NS + s) * EPT

    def body(i, _):
        off = pl.multiple_of(base + i * K, 8)
        pltpu.sync_copy(dst_hbm.at[pl.ds(off, K)], dst_v)
        pltpu.sync_copy(ones_v, acc.at[dst_v], add=True)
        return 0

    lax.fori_loop(0, NITER, body, 0)
    plsc.subcore_barrier()
    pltpu.sync_copy(acc.at[pl.ds(s * RPT, RPT)],
                    out_hbm.at[c, pl.ds(s * RPT, RPT)])


@functools.partial(
    pl.kernel,
    out_type=jax.ShapeDtypeStruct((NC, NPAD, D), f32),
    mesh=_mesh,
    scratch_types=[
        pltpu.VMEM((K,), jnp.int32),
        pltpu.VMEM((K,), jnp.int32),
        pltpu.VMEM((K, D), f32),
        pltpu.VMEM((128, D), f32),
        pltpu.VMEM_SHARED((NPAD, D), f32),
        pltpu.SemaphoreType.DMA,
    ],
)
def _agg_kernel(g_hbm, src_hbm, dst_hbm, out_hbm,
                src_v, dst_v, rows_v, zbuf, acc, sem):
    c = lax.axis_index("c")
    s = lax.axis_index("s")
    zeros = jnp.zeros((16,), f32)

    def fill(i, _):
        for j in range(D // 16):
            zbuf[i, pl.ds(j * 16, 16)] = zeros
        return 0

    lax.fori_loop(0, 128, fill, 0)
    for k in range(RPT // 128):
        pltpu.sync_copy(zbuf, acc.at[pl.ds(s * RPT + k * 128, 128)])
    plsc.subcore_barrier()

    base = (c * NS + s) * EPT

    def body(i, _):
        off = pl.multiple_of(base + i * K, 8)
        pltpu.sync_copy(src_hbm.at[pl.ds(off, K)], src_v)
        pltpu.sync_copy(dst_hbm.at[pl.ds(off, K)], dst_v)
        pltpu.async_copy(g_hbm.at[src_v], rows_v, sem).wait()
        pltpu.sync_copy(rows_v, acc.at[dst_v], add=True)
        return 0

    lax.fori_loop(0, NITER, body, 0)
    plsc.subcore_barrier()
    pltpu.sync_copy(acc.at[pl.ds(s * RPT, RPT)],
                    out_hbm.at[c, pl.ds(s * RPT, RPT)])


def _dot(a, b):
    return jnp.dot(a, b, precision=lax.Precision.HIGHEST,
                   preferred_element_type=f32)


def _dis_col(deg_ref):
    p = deg_ref[0, :N, :] + deg_ref[1, :N, :]
    deg = jnp.sum(p, axis=1, keepdims=True) * (1.0 / DW) + 1.0
    return lax.rsqrt(deg)


def _tc_first_body(deg_ref, x_ref, w_ref, u_ref, g_ref):
    dis = _dis_col(deg_ref)
    u = _dot(x_ref[...], w_ref[...])
    u_ref[...] = u
    g_ref[...] = u * dis


def _tc_mid_body(deg_ref, s_ref, u_ref, b_ref, w_ref, u2_ref, g2_ref):
    dis = _dis_col(deg_ref)
    ssum = s_ref[0, :N, :] + s_ref[1, :N, :]
    h = jnp.maximum(ssum * dis + u_ref[...] * (dis * dis) + b_ref[...], 0.0)
    u2 = _dot(h, w_ref[...])
    u2_ref[...] = u2
    g2_ref[...] = u2 * dis


def _tc_final_body(deg_ref, s_ref, u_ref, b_ref, batch_ref, wfc_ref, bfc_ref,
                   o_ref):
    dis = _dis_col(deg_ref)
    ssum = s_ref[0, :N, :] + s_ref[1, :N, :]
    h = jnp.maximum(ssum * dis + u_ref[...] * (dis * dis) + b_ref[...], 0.0)
    seg = batch_ref[...]
    sel = (seg == lax.broadcasted_iota(jnp.int32, (G, N), 0)).astype(f32)
    psum = _dot(sel, h)
    cnt = jnp.sum(sel, axis=1, keepdims=True)
    pooled = psum / jnp.maximum(cnt, 1.0)
    o_ref[...] = _dot(pooled, wfc_ref[...]) + bfc_ref[...]


_tc_first = pl.pallas_call(
    _tc_first_body,
    out_shape=(jax.ShapeDtypeStruct((N, D), f32),
               jax.ShapeDtypeStruct((N, D), f32)),
)

_tc_mid = pl.pallas_call(
    _tc_mid_body,
    out_shape=(jax.ShapeDtypeStruct((N, D), f32),
               jax.ShapeDtypeStruct((N, D), f32)),
)

_tc_final = pl.pallas_call(
    _tc_final_body,
    out_shape=jax.ShapeDtypeStruct((G, 4), f32),
)


def kernel(x, edge_index, batch, W1, b1, W2, b2, W3, b3, W4, b4, Wfc, bfc):
    src = edge_index[0]
    dst = edge_index[1]
    batch2 = batch.reshape(1, N)
    b1r, b2r, b3r, b4r = (b.reshape(1, D) for b in (b1, b2, b3, b4))
    bfcr = bfc.reshape(1, 4)

    degp = _deg_kernel(dst)
    u1, g1 = _tc_first(degp, x, W1)
    s1 = _agg_kernel(g1, src, dst)
    u2, g2 = _tc_mid(degp, s1, u1, b1r, W2)
    s2 = _agg_kernel(g2, src, dst)
    u3, g3 = _tc_mid(degp, s2, u2, b2r, W3)
    s3 = _agg_kernel(g3, src, dst)
    u4, g4 = _tc_mid(degp, s3, u3, b3r, W4)
    s4 = _agg_kernel(g4, src, dst)
    return _tc_final(degp, s4, u4, b4r, batch2, Wfc, bfcr)


# speedup vs baseline: 10.2760x; 10.2760x over previous
import functools

import jax
import jax.numpy as jnp
from jax import lax
from jax.experimental import pallas as pl
from jax.experimental.pallas import tpu as pltpu
from jax.experimental.pallas import tpu_sc as plsc

N = 10000
E = 320000
D = 128
G = 64
NC = 2
NS = 16
NW = NC * NS
EPT = E // NW
K = 80
NITER = EPT // K
NPAD = 10240
RPT = NPAD // NS
DW = 16

f32 = jnp.float32
_mesh = plsc.VectorSubcoreMesh(core_axis_name="c", subcore_axis_name="s")


@functools.partial(
    pl.kernel,
    out_type=jax.ShapeDtypeStruct((NC, NPAD, DW), f32),
    mesh=_mesh,
    scratch_types=[
        pltpu.VMEM((K,), jnp.int32),
        pltpu.VMEM((K, DW), f32),
        pltpu.VMEM((128, DW), f32),
        pltpu.VMEM_SHARED((NPAD, DW), f32),
    ],
)
def _deg_kernel(dst_hbm, out_hbm, dst_v, ones_v, zbuf, acc):
    c = lax.axis_index("c")
    s = lax.axis_index("s")
    ones = jnp.full((16,), 1.0, f32)
    zeros = jnp.zeros((16,), f32)

    def fill(i, _):
        ones_v[i, :] = ones
        zbuf[i, :] = zeros
        return 0

    lax.fori_loop(0, K, fill, 0)

    def fill2(i, _):
        zbuf[i, :] = zeros
        return 0

    lax.fori_loop(K, 128, fill2, 0)
    for k in range(RPT // 128):
        pltpu.sync_copy(zbuf, acc.at[pl.ds(s * RPT + k * 128, 128)])
    plsc.subcore_barrier()

    base = (c * NS + s) * EPT

    def body(i, _):
        off = pl.multiple_of(base + i * K, 8)
        pltpu.sync_copy(dst_hbm.at[pl.ds(off, K)], dst_v)
        pltpu.sync_copy(ones_v, acc.at[dst_v], add=True)
        return 0

    lax.fori_loop(0, NITER, body, 0)
    plsc.subcore_barrier()
    pltpu.sync_copy(acc.at[pl.ds(s * RPT, RPT)],
                    out_hbm.at[c, pl.ds(s * RPT, RPT)])


@functools.partial(
    pl.kernel,
    out_type=jax.ShapeDtypeStruct((NC, NPAD, D), f32),
    mesh=_mesh,
    scratch_types=[
        pltpu.VMEM((K,), jnp.int32),
        pltpu.VMEM((K,), jnp.int32),
        pltpu.VMEM((K, D), f32),
        pltpu.VMEM((128, D), f32),
        pltpu.VMEM_SHARED((NPAD, D), f32),
        pltpu.SemaphoreType.DMA,
    ],
)
def _agg_kernel(g_hbm, src_hbm, dst_hbm, out_hbm,
                src_v, dst_v, rows_v, zbuf, acc, sem):
    c = lax.axis_index("c")
    s = lax.axis_index("s")
    zeros = jnp.zeros((16,), f32)

    def fill(i, _):
        for j in range(D // 16):
            zbuf[i, pl.ds(j * 16, 16)] = zeros
        return 0

    lax.fori_loop(0, 128, fill, 0)
    for k in range(RPT // 128):
        pltpu.sync_copy(zbuf, acc.at[pl.ds(s * RPT + k * 128, 128)])
    plsc.subcore_barrier()

    base = (c * NS + s) * EPT

    def body(i, _):
        off = pl.multiple_of(base + i * K, 8)
        pltpu.sync_copy(src_hbm.at[pl.ds(off, K)], src_v)
        pltpu.sync_copy(dst_hbm.at[pl.ds(off, K)], dst_v)
        pltpu.async_copy(g_hbm.at[src_v], rows_v, sem).wait()
        pltpu.sync_copy(rows_v, acc.at[dst_v], add=True)
        return 0

    lax.fori_loop(0, NITER, body, 0)
    plsc.subcore_barrier()
    pltpu.sync_copy(acc.at[pl.ds(s * RPT, RPT)],
                    out_hbm.at[c, pl.ds(s * RPT, RPT)])


BN = 2000
NB = N // BN

def _dot(a, b):
    return jnp.dot(a, b, precision=lax.Precision.HIGHEST,
                   preferred_element_type=f32)


def _dis_col(deg_ref):
    p = deg_ref[0] + deg_ref[1]
    deg = jnp.sum(p, axis=1, keepdims=True) * (1.0 / DW) + 1.0
    return lax.rsqrt(deg)


def _tc_first_body(deg_ref, x_ref, w_ref, u_ref, g_ref):
    dis = _dis_col(deg_ref)
    u = _dot(x_ref[...], w_ref[...])
    u_ref[...] = u
    g_ref[...] = u * dis


def _tc_mid_body(deg_ref, s_ref, u_ref, b_ref, w_ref, u2_ref, g2_ref):
    dis = _dis_col(deg_ref)
    ssum = s_ref[0] + s_ref[1]
    h = jnp.maximum(ssum * dis + u_ref[...] * (dis * dis) + b_ref[...], 0.0)
    u2 = _dot(h, w_ref[...])
    u2_ref[...] = u2
    g2_ref[...] = u2 * dis


def _tc_final_body(deg_ref, s_ref, u_ref, b_ref, batch_ref, wfc_ref, bfc_ref,
                   o_ref, psum_acc, cnt_acc):
    i = pl.program_id(0)
    dis = _dis_col(deg_ref)
    ssum = s_ref[0] + s_ref[1]
    h = jnp.maximum(ssum * dis + u_ref[...] * (dis * dis) + b_ref[...], 0.0)
    seg = batch_ref[0]
    sel = (seg == lax.broadcasted_iota(jnp.int32, (G, BN), 0)).astype(f32)
    psum = _dot(sel, h)
    cnt = jnp.sum(sel, axis=1, keepdims=True) * jnp.ones((1, 128), f32)

    @pl.when(i == 0)
    def _():
        psum_acc[...] = psum
        cnt_acc[...] = cnt

    @pl.when(i > 0)
    def _():
        psum_acc[...] += psum
        cnt_acc[...] += cnt

    @pl.when(i == NB - 1)
    def _():
        pooled = psum_acc[...] / jnp.maximum(cnt_acc[...], 1.0)
        o_ref[...] = _dot(pooled, wfc_ref[...]) + bfc_ref[...]


_deg_spec = pl.BlockSpec((NC, BN, DW), lambda i: (0, i, 0))
_s_spec = pl.BlockSpec((NC, BN, D), lambda i: (0, i, 0))
_row_spec = pl.BlockSpec((BN, D), lambda i: (i, 0))
_w_spec = pl.BlockSpec((D, D), lambda i: (0, 0))
_b_spec = pl.BlockSpec((1, D), lambda i: (0, 0))

_tc_first = pl.pallas_call(
    _tc_first_body,
    grid=(NB,),
    in_specs=[_deg_spec, _row_spec, _w_spec],
    out_specs=(_row_spec, _row_spec),
    out_shape=(jax.ShapeDtypeStruct((N, D), f32),
               jax.ShapeDtypeStruct((N, D), f32)),
)

_tc_mid = pl.pallas_call(
    _tc_mid_body,
    grid=(NB,),
    in_specs=[_deg_spec, _s_spec, _row_spec, _b_spec, _w_spec],
    out_specs=(_row_spec, _row_spec),
    out_shape=(jax.ShapeDtypeStruct((N, D), f32),
               jax.ShapeDtypeStruct((N, D), f32)),
)

_tc_final = pl.pallas_call(
    _tc_final_body,
    grid=(NB,),
    in_specs=[_deg_spec, _s_spec, _row_spec, _b_spec,
              pl.BlockSpec((1, 1, BN), lambda i: (i, 0, 0)),
              pl.BlockSpec((D, 4), lambda i: (0, 0)),
              pl.BlockSpec((1, 4), lambda i: (0, 0))],
    out_specs=pl.BlockSpec((G, 4), lambda i: (0, 0)),
    out_shape=jax.ShapeDtypeStruct((G, 4), f32),
    scratch_shapes=[pltpu.VMEM((G, D), f32), pltpu.VMEM((G, D), f32)],
)


def kernel(x, edge_index, batch, W1, b1, W2, b2, W3, b3, W4, b4, Wfc, bfc):
    src = edge_index[0]
    dst = edge_index[1]
    batch2 = batch.reshape(NB, 1, BN)
    b1r, b2r, b3r, b4r = (b.reshape(1, D) for b in (b1, b2, b3, b4))
    bfcr = bfc.reshape(1, 4)

    degp = _deg_kernel(dst)
    u1, g1 = _tc_first(degp, x, W1)
    s1 = _agg_kernel(g1, src, dst)
    u2, g2 = _tc_mid(degp, s1, u1, b1r, W2)
    s2 = _agg_kernel(g2, src, dst)
    u3, g3 = _tc_mid(degp, s2, u2, b2r, W3)
    s3 = _agg_kernel(g3, src, dst)
    u4, g4 = _tc_mid(degp, s3, u3, b3r, W4)
    s4 = _agg_kernel(g4, src, dst)
    return _tc_final(degp, s4, u4, b4r, batch2, Wfc, bfcr)
